# Initial kernel scaffold; baseline (speedup 1.0000x reference)
import jax
import jax.numpy as jnp
from jax.experimental import pallas as pl


def kernel(inputs, W):
    raise NotImplementedError("write your pallas kernel here")


# speedup vs baseline: 1.4817x; 1.4817x over previous
import functools

import jax
import jax.numpy as jnp
from jax import lax
from jax.experimental import pallas as pl
from jax.experimental.pallas import tpu as pltpu
from jax.experimental.pallas import tpu_sc as plsc

K = 8192
CODE_DIM = 256
B = 16
HW = 1024
N = B * HW


TM = 256
GRID = N // TM


_SEG = (0, 2816, 5632, 8192)


def _argmax_body(x_ref, w_ref, idx_ref):
    x = x_ref[...].astype(jnp.bfloat16)
    w = w_ref[...].astype(jnp.bfloat16)
    logits = lax.dot_general(
        x, w, (((1,), (1,)), ((), ())),
        preferred_element_type=jnp.float32,
    )

    def seg_max_arg(lo, hi):
        seg = logits[:, lo:hi]
        m = jnp.max(seg, axis=1, keepdims=True)
        iota = lax.broadcasted_iota(jnp.int32, seg.shape, 1)
        idx = jnp.min(jnp.where(seg == m, iota, K), axis=1) + lo
        return m[:, 0], idx

    v0, i0 = seg_max_arg(_SEG[0], _SEG[1])
    v1, i1 = seg_max_arg(_SEG[1], _SEG[2])
    v2, i2 = seg_max_arg(_SEG[2], _SEG[3])
    acc_v = v0.astype(jnp.bfloat16).astype(jnp.float32)
    acc_i = i0
    w1 = v1 > acc_v
    acc_v = jnp.where(w1, v1.astype(jnp.bfloat16).astype(jnp.float32), acc_v)
    acc_i = jnp.where(w1, i1, acc_i)
    w2 = v2 > acc_v
    acc_i = jnp.where(w2, i2, acc_i)
    idx_ref[0, 0, :] = acc_i


def _compute_indices(x):
    return pl.pallas_call(
        _argmax_body,
        grid=(GRID,),
        in_specs=[
            pl.BlockSpec((TM, CODE_DIM), lambda i: (i, 0)),
            pl.BlockSpec((K, CODE_DIM), lambda i: (0, 0)),
        ],
        out_specs=pl.BlockSpec((1, 1, TM), lambda i: (i, 0, 0)),
        out_shape=jax.ShapeDtypeStruct((GRID, 1, TM), jnp.int32),
    )


NC, NS = 2, 16
NW = NC * NS
B_PER_W = N // NW
CHUNK = 128
NCHUNK = B_PER_W // CHUNK


def _gather_body(idx_hbm, table_hbm, out_hbm, idx_v, rows_v, sem):
    wid = lax.axis_index("s") * NC + lax.axis_index("c")
    base = wid * B_PER_W
    for i in range(NCHUNK):
        off = base + i * CHUNK
        pltpu.sync_copy(idx_hbm.at[pl.ds(off, CHUNK)], idx_v)
        pltpu.async_copy(table_hbm.at[idx_v], rows_v, sem).wait()
        pltpu.sync_copy(rows_v, out_hbm.at[pl.ds(off, CHUNK)])


@functools.cache
def _gather():
    return pl.kernel(
        _gather_body,
        out_type=jax.ShapeDtypeStruct((N, CODE_DIM), jnp.float32),
        mesh=plsc.VectorSubcoreMesh(core_axis_name="c", subcore_axis_name="s"),
        scratch_types=[
            pltpu.VMEM((CHUNK,), jnp.int32),
            pltpu.VMEM((CHUNK, CODE_DIM), jnp.float32),
            pltpu.SemaphoreType.DMA,
        ],
    )


def kernel(inputs, W):
    x = inputs.reshape(N, CODE_DIM)
    idx = _compute_indices(x)(x, W).reshape(N)
    embed = _gather()(idx, W)
    return embed.reshape(B, HW, CODE_DIM)


# speedup vs baseline: 1.7264x; 1.1652x over previous
import functools

import jax
import jax.numpy as jnp
from jax import lax
from jax.experimental import pallas as pl
from jax.experimental.pallas import tpu as pltpu
from jax.experimental.pallas import tpu_sc as plsc

K = 8192
CODE_DIM = 256
B = 16
HW = 1024
N = B * HW


TM = 256
GRID = N // TM


_SEG = (0, 2816, 5632, 8192)


def _argmax_body(x_ref, w_ref, idx_ref):
    x = x_ref[...].astype(jnp.bfloat16)
    w = w_ref[...].astype(jnp.bfloat16)
    logits = lax.dot_general(
        x, w, (((1,), (1,)), ((), ())),
        preferred_element_type=jnp.float32,
    )

    lane = lax.broadcasted_iota(jnp.int32, (TM, 128), 1)

    def seg_max_arg(lo, hi):
        seg = logits[:, lo:hi]
        m = jnp.max(seg, axis=1, keepdims=True)
        nch = (hi - lo) // 128
        macc = jnp.full((TM, 128), K, jnp.int32)
        for c in reversed(range(nch)):
            eq = seg[:, c * 128:(c + 1) * 128] == m
            macc = jnp.where(eq, jnp.int32(c), macc)
        kk = jnp.where(macc == K, jnp.int32(K), macc * 128 + lane)
        idx = jnp.min(kk, axis=1) + lo
        return m[:, 0], idx

    v0, i0 = seg_max_arg(_SEG[0], _SEG[1])
    v1, i1 = seg_max_arg(_SEG[1], _SEG[2])
    v2, i2 = seg_max_arg(_SEG[2], _SEG[3])
    acc_v = v0.astype(jnp.bfloat16).astype(jnp.float32)
    acc_i = i0
    w1 = v1 > acc_v
    acc_v = jnp.where(w1, v1.astype(jnp.bfloat16).astype(jnp.float32), acc_v)
    acc_i = jnp.where(w1, i1, acc_i)
    w2 = v2 > acc_v
    acc_i = jnp.where(w2, i2, acc_i)
    idx_ref[0, 0, :] = acc_i


def _compute_indices(x):
    return pl.pallas_call(
        _argmax_body,
        grid=(GRID,),
        in_specs=[
            pl.BlockSpec((TM, CODE_DIM), lambda i: (i, 0)),
            pl.BlockSpec((K, CODE_DIM), lambda i: (0, 0)),
        ],
        out_specs=pl.BlockSpec((1, 1, TM), lambda i: (i, 0, 0)),
        out_shape=jax.ShapeDtypeStruct((GRID, 1, TM), jnp.int32),
    )


NC, NS = 2, 16
NW = NC * NS
B_PER_W = N // NW
CHUNK = 128
NCHUNK = B_PER_W // CHUNK


def _gather_body(idx_hbm, table_hbm, out_hbm, idx_v, rows_v, sem):
    wid = lax.axis_index("s") * NC + lax.axis_index("c")
    base = wid * B_PER_W
    for i in range(NCHUNK):
        off = base + i * CHUNK
        pltpu.sync_copy(idx_hbm.at[pl.ds(off, CHUNK)], idx_v)
        pltpu.async_copy(table_hbm.at[idx_v], rows_v, sem).wait()
        pltpu.sync_copy(rows_v, out_hbm.at[pl.ds(off, CHUNK)])


@functools.cache
def _gather():
    return pl.kernel(
        _gather_body,
        out_type=jax.ShapeDtypeStruct((N, CODE_DIM), jnp.float32),
        mesh=plsc.VectorSubcoreMesh(core_axis_name="c", subcore_axis_name="s"),
        scratch_types=[
            pltpu.VMEM((CHUNK,), jnp.int32),
            pltpu.VMEM((CHUNK, CODE_DIM), jnp.float32),
            pltpu.SemaphoreType.DMA,
        ],
    )


def kernel(inputs, W):
    x = inputs.reshape(N, CODE_DIM)
    idx = _compute_indices(x)(x, W).reshape(N)
    embed = _gather()(idx, W)
    return embed.reshape(B, HW, CODE_DIM)


# speedup vs baseline: 1.8735x; 1.0852x over previous
import functools

import jax
import jax.numpy as jnp
from jax import lax
from jax.experimental import pallas as pl
from jax.experimental.pallas import tpu as pltpu
from jax.experimental.pallas import tpu_sc as plsc

K = 8192
CODE_DIM = 256
B = 16
HW = 1024
N = B * HW


TM = 512
GRID = N // TM


_SEG = (0, 2816, 5632, 8192)


def _argmax_body(x_ref, w_ref, idx_ref):
    x = x_ref[...].astype(jnp.bfloat16)
    w = w_ref[...].astype(jnp.bfloat16)
    logits = lax.dot_general(
        x, w, (((1,), (1,)), ((), ())),
        preferred_element_type=jnp.float32,
    )

    lane = lax.broadcasted_iota(jnp.int32, (TM, 128), 1)

    def seg_max_arg(lo, hi):
        seg = logits[:, lo:hi]
        m = jnp.max(seg, axis=1, keepdims=True)
        nch = (hi - lo) // 128
        macc = jnp.full((TM, 128), K, jnp.int32)
        for c in reversed(range(nch)):
            eq = seg[:, c * 128:(c + 1) * 128] == m
            macc = jnp.where(eq, jnp.int32(c), macc)
        kk = jnp.where(macc == K, jnp.int32(K), macc * 128 + lane)
        idx = jnp.min(kk, axis=1) + lo
        return m[:, 0], idx

    v0, i0 = seg_max_arg(_SEG[0], _SEG[1])
    v1, i1 = seg_max_arg(_SEG[1], _SEG[2])
    v2, i2 = seg_max_arg(_SEG[2], _SEG[3])
    acc_v = v0.astype(jnp.bfloat16).astype(jnp.float32)
    acc_i = i0
    w1 = v1 > acc_v
    acc_v = jnp.where(w1, v1.astype(jnp.bfloat16).astype(jnp.float32), acc_v)
    acc_i = jnp.where(w1, i1, acc_i)
    w2 = v2 > acc_v
    acc_i = jnp.where(w2, i2, acc_i)
    idx_ref[0, 0, :] = acc_i


def _compute_indices(x):
    return pl.pallas_call(
        _argmax_body,
        grid=(GRID,),
        in_specs=[
            pl.BlockSpec((TM, CODE_DIM), lambda i: (i, 0)),
            pl.BlockSpec((K, CODE_DIM), lambda i: (0, 0)),
        ],
        out_specs=pl.BlockSpec((1, 1, TM), lambda i: (i, 0, 0)),
        out_shape=jax.ShapeDtypeStruct((GRID, 1, TM), jnp.int32),
    )


NC, NS = 2, 16
NW = NC * NS
B_PER_W = N // NW
CHUNK = 128
NCHUNK = B_PER_W // CHUNK


def _gather_body(idx_hbm, table_hbm, out_hbm, idx_v, rows_v, sem):
    wid = lax.axis_index("s") * NC + lax.axis_index("c")
    base = wid * B_PER_W
    for i in range(NCHUNK):
        off = base + i * CHUNK
        pltpu.sync_copy(idx_hbm.at[pl.ds(off, CHUNK)], idx_v)
        pltpu.async_copy(table_hbm.at[idx_v], rows_v, sem).wait()
        pltpu.sync_copy(rows_v, out_hbm.at[pl.ds(off, CHUNK)])


@functools.cache
def _gather():
    return pl.kernel(
        _gather_body,
        out_type=jax.ShapeDtypeStruct((N, CODE_DIM), jnp.float32),
        mesh=plsc.VectorSubcoreMesh(core_axis_name="c", subcore_axis_name="s"),
        scratch_types=[
            pltpu.VMEM((CHUNK,), jnp.int32),
            pltpu.VMEM((CHUNK, CODE_DIM), jnp.float32),
            pltpu.SemaphoreType.DMA,
        ],
    )


def kernel(inputs, W):
    x = inputs.reshape(N, CODE_DIM)
    idx = _compute_indices(x)(x, W).reshape(N)
    embed = _gather()(idx, W)
    return embed.reshape(B, HW, CODE_DIM)
